# baseline (speedup 1.0000x reference)
import functools

import jax
import jax.numpy as jnp
from jax.experimental import pallas as pl
from jax.experimental.pallas import tpu as pltpu

_MIB = 1024 * 1024


def _se_nhwc_kernel(x_ref, w1_hbm, w2t_hbm, o_ref,
                    w1_vmem, w2t_vmem, sem1, sem2, *, inv_hw):
    i = pl.program_id(0)

    @pl.when(i == 0)
    def _():
        pltpu.make_async_copy(w1_hbm, w1_vmem, sem1).start()
        pltpu.make_async_copy(w2t_hbm, w2t_vmem, sem2).start()

    x = x_ref[...]

    pooled = jnp.sum(x, axis=(1, 2), dtype=jnp.float32) * inv_hw

    @pl.when(i == 0)
    def _():
        pltpu.make_async_copy(w1_hbm, w1_vmem, sem1).wait()
        pltpu.make_async_copy(w2t_hbm, w2t_vmem, sem2).wait()

    y1 = jax.lax.dot_general(pooled, w1_vmem[...],
                             (((1,), (1,)), ((), ())),
                             preferred_element_type=jnp.float32)
    y1 = jnp.clip(y1, 0.0, 6.0)
    y2 = jax.lax.dot_general(y1, w2t_vmem[...],
                             (((1,), (0,)), ((), ())),
                             preferred_element_type=jnp.float32)
    scale = jax.nn.sigmoid(y2)

    o_ref[...] = x * scale[:, None, None, :].astype(x.dtype)


def kernel(x, w1, w2):
    B, C, H, W = x.shape
    R = w1.shape[0]
    HW = H * W
    itemsize = jnp.dtype(x.dtype).itemsize

    xt = jnp.transpose(x, (0, 2, 3, 1))
    w2t = jnp.transpose(w2, (1, 0))

    slab = H * W * C * itemsize
    bb = 1
    for cand in (8, 4, 2, 1):
        if B % cand == 0 and cand * slab <= 8 * _MIB:
            bb = cand
            break

    body = functools.partial(_se_nhwc_kernel, inv_hw=1.0 / float(HW))
    cost = pl.CostEstimate(
        flops=int(2 * B * C * HW + 4 * B * C * R),
        transcendentals=int(B * C),
        bytes_accessed=int(2 * B * C * HW * itemsize + (w1.size + w2.size) * 4),
    )

    out_t = pl.pallas_call(
        body,
        out_shape=jax.ShapeDtypeStruct((B, H, W, C), x.dtype),
        grid=(B // bb,),
        in_specs=[
            pl.BlockSpec((bb, H, W, C), lambda i: (i, 0, 0, 0)),
            pl.BlockSpec(memory_space=pltpu.MemorySpace.HBM),
            pl.BlockSpec(memory_space=pltpu.MemorySpace.HBM),
        ],
        out_specs=pl.BlockSpec((bb, H, W, C), lambda i: (i, 0, 0, 0)),
        scratch_shapes=[
            pltpu.VMEM((R, C), jnp.float32),
            pltpu.VMEM((R, C), jnp.float32),
            pltpu.SemaphoreType.DMA,
            pltpu.SemaphoreType.DMA,
        ],
        compiler_params=pltpu.CompilerParams(
            dimension_semantics=("arbitrary",),
            vmem_limit_bytes=int(min(56 * _MIB, 4 * bb * slab + 12 * _MIB)),
        ),
        cost_estimate=cost,
    )(xt, w1, w2t)

    return jnp.transpose(out_t, (0, 3, 1, 2))


# speedup vs baseline: 1.0132x; 1.0132x over previous
import functools

import jax
import jax.numpy as jnp
from jax.experimental import pallas as pl
from jax.experimental.pallas import tpu as pltpu

_MIB = 1024 * 1024


def _se_nhwc_kernel(x_ref, w1_ref, w2t_ref, o_ref, *, inv_hw):
    x = x_ref[...]

    pooled = jnp.sum(x, axis=(1, 2), dtype=jnp.float32) * inv_hw

    y1 = jax.lax.dot_general(pooled, w1_ref[...],
                             (((1,), (1,)), ((), ())),
                             preferred_element_type=jnp.float32)
    y1 = jnp.clip(y1, 0.0, 6.0)
    y2 = jax.lax.dot_general(y1, w2t_ref[...],
                             (((1,), (0,)), ((), ())),
                             preferred_element_type=jnp.float32)
    scale = jax.nn.sigmoid(y2)

    o_ref[...] = x * scale[:, None, None, :].astype(x.dtype)


def kernel(x, w1, w2):
    B, C, H, W = x.shape
    R = w1.shape[0]
    HW = H * W
    itemsize = jnp.dtype(x.dtype).itemsize

    xt = jnp.transpose(x, (0, 2, 3, 1))
    w2t = jnp.transpose(w2, (1, 0))

    slab = H * W * C * itemsize
    bb = 1
    for cand in (8, 4, 2, 1):
        if B % cand == 0 and cand * slab <= 8 * _MIB:
            bb = cand
            break

    body = functools.partial(_se_nhwc_kernel, inv_hw=1.0 / float(HW))
    cost = pl.CostEstimate(
        flops=int(2 * B * C * HW + 4 * B * C * R),
        transcendentals=int(B * C),
        bytes_accessed=int(2 * B * C * HW * itemsize + (w1.size + w2.size) * 4),
    )

    out_t = pl.pallas_call(
        body,
        out_shape=jax.ShapeDtypeStruct((B, H, W, C), x.dtype),
        grid=(B // bb,),
        in_specs=[
            pl.BlockSpec((bb, H, W, C), lambda i: (i, 0, 0, 0)),
            pl.BlockSpec((R, C), lambda i: (0, 0)),
            pl.BlockSpec((R, C), lambda i: (0, 0)),
        ],
        out_specs=pl.BlockSpec((bb, H, W, C), lambda i: (i, 0, 0, 0)),
        compiler_params=pltpu.CompilerParams(
            dimension_semantics=("arbitrary",),
            vmem_limit_bytes=int(min(56 * _MIB, 4 * bb * slab + 4 * _MIB)),
        ),
        cost_estimate=cost,
    )(xt, w1, w2t)

    return jnp.transpose(out_t, (0, 3, 1, 2))


# speedup vs baseline: 1.0288x; 1.0154x over previous
import functools

import jax
import jax.numpy as jnp
from jax.experimental import pallas as pl
from jax.experimental.pallas import tpu as pltpu

_MIB = 1024 * 1024


def _se_nhwc_kernel(x_ref, w_ref, o_ref, *, inv_hw, r):
    x = x_ref[...]

    pooled = jnp.sum(x, axis=(1, 2), dtype=jnp.float32) * inv_hw

    y1 = jax.lax.dot_general(pooled, w_ref[:r, :],
                             (((1,), (1,)), ((), ())),
                             preferred_element_type=jnp.float32)
    y1 = jnp.clip(y1, 0.0, 6.0)
    y2 = jax.lax.dot_general(y1, w_ref[r:, :],
                             (((1,), (0,)), ((), ())),
                             preferred_element_type=jnp.float32)
    scale = jax.nn.sigmoid(y2)

    o_ref[...] = x * scale[:, None, None, :].astype(x.dtype)


def kernel(x, w1, w2):
    B, C, H, W = x.shape
    R = w1.shape[0]
    HW = H * W
    itemsize = jnp.dtype(x.dtype).itemsize

    xt = jnp.transpose(x, (0, 2, 3, 1))
    w_cat = jnp.concatenate([w1, jnp.transpose(w2, (1, 0))], axis=0)

    slab = H * W * C * itemsize
    bb = 1
    for cand in (8, 4, 2, 1):
        if B % cand == 0 and cand * slab <= 8 * _MIB:
            bb = cand
            break

    body = functools.partial(_se_nhwc_kernel, inv_hw=1.0 / float(HW), r=R)
    cost = pl.CostEstimate(
        flops=int(2 * B * C * HW + 4 * B * C * R),
        transcendentals=int(B * C),
        bytes_accessed=int(2 * B * C * HW * itemsize + (w1.size + w2.size) * 4),
    )

    out_t = pl.pallas_call(
        body,
        out_shape=jax.ShapeDtypeStruct((B, H, W, C), x.dtype),
        grid=(B // bb,),
        in_specs=[
            pl.BlockSpec((bb, H, W, C), lambda i: (i, 0, 0, 0)),
            pl.BlockSpec((2 * R, C), lambda i: (0, 0)),
        ],
        out_specs=pl.BlockSpec((bb, H, W, C), lambda i: (i, 0, 0, 0)),
        compiler_params=pltpu.CompilerParams(
            dimension_semantics=("arbitrary",),
            vmem_limit_bytes=int(min(56 * _MIB, 4 * bb * slab + 4 * _MIB)),
        ),
        cost_estimate=cost,
    )(xt, w_cat)

    return jnp.transpose(out_t, (0, 3, 1, 2))
